# baseline (speedup 1.0000x reference)
import functools

import jax
import jax.numpy as jnp
from jax import lax
from jax.experimental import pallas as pl
from jax.experimental.pallas import tpu as pltpu
from jax.experimental.pallas import tpu_sc as plsc

MIN_NORM = 1e-15
NC, NS = 2, 16
NW = NC * NS
LANES = 16
K = 80
LN2 = 0.6931471805599453


def _artanh(z):
  z = jnp.clip(z, -1.0 + 1e-7, 1.0 - 1e-7)
  return 0.5 * jnp.log((1.0 + z) / (1.0 - z))


def _rownorm(v):
  return jnp.sqrt(jnp.sum(v * v, axis=-1, keepdims=True))


def _logmap0(p):
  n = jnp.maximum(_rownorm(p), MIN_NORM)
  return p / n * _artanh(n)


def _expmap0(u):
  n = jnp.maximum(_rownorm(u), MIN_NORM)
  return jnp.tanh(n) * u / n


def _proj(x):
  n = jnp.maximum(_rownorm(x), MIN_NORM)
  maxn = 1.0 - 4e-3
  return jnp.where(n > maxn, x / n * maxn, x)


def _vperm(v, idx):
  dnums = lax.GatherDimensionNumbers(
      offset_dims=(), collapsed_slice_dims=(0,), start_index_map=(0,))
  return lax.gather(v, idx[:, None], dnums, (1,),
                    mode=lax.GatherScatterMode.PROMISE_IN_BOUNDS)


def _sw_rsqrt(x):
  i = lax.bitcast_convert_type(x, jnp.int32)
  i = 0x5F3759DF - lax.shift_right_logical(i, 1)
  y = lax.bitcast_convert_type(i, jnp.float32)
  for _ in range(3):
    y = y * (1.5 - 0.5 * x * y * y)
  return y


def _sw_sqrt(x):
  return jnp.where(x < 1e-30, 0.0, x * _sw_rsqrt(x))


def _sw_ln(y):
  bits = lax.bitcast_convert_type(y, jnp.int32)
  e = lax.shift_right_logical(bits, 23) - 127
  m = lax.bitcast_convert_type((bits & 0x007FFFFF) | 0x3F800000, jnp.float32)
  big = m > 1.4142135
  m = jnp.where(big, m * 0.5, m)
  e = (e + jnp.where(big, 1, 0)).astype(jnp.float32)
  s = (m - 1.0) / (m + 1.0)
  s2 = s * s
  p = 1.0 / 9.0
  p = p * s2 + 1.0 / 7.0
  p = p * s2 + 1.0 / 5.0
  p = p * s2 + 1.0 / 3.0
  p = p * s2 + 1.0
  return e * LN2 + 2.0 * s * p


def _pack_rows(x):
  d = x.shape[-1]
  xb = x.astype(jnp.bfloat16)
  lo = lax.bitcast_convert_type(xb[:, :d // 2], jnp.uint16).astype(jnp.uint32)
  hi = lax.bitcast_convert_type(xb[:, d // 2:], jnp.uint16).astype(jnp.uint32)
  return lax.bitcast_convert_type((hi << 16) | lo, jnp.int32)


def _prep_body(x_ref, nsq_ref, g_ref, xp_ref):
  x = x_ref[...]
  sq = jnp.sum(x * x, axis=-1, keepdims=True)
  nsq_ref[...] = sq
  n = jnp.maximum(jnp.sqrt(sq), MIN_NORM)
  g_ref[...] = _artanh(n) / n
  xp_ref[...] = _pack_rows(x)


@functools.lru_cache(maxsize=None)
def _make_prep(n_nodes, d, bn):
  return pl.pallas_call(
      _prep_body,
      grid=(n_nodes // bn,),
      in_specs=[pl.BlockSpec((bn, d), lambda i: (i, 0))],
      out_specs=[
          pl.BlockSpec((bn, 1), lambda i: (i, 0)),
          pl.BlockSpec((bn, 1), lambda i: (i, 0)),
          pl.BlockSpec((bn, d // 2), lambda i: (i, 0)),
      ],
      out_shape=[
          jax.ShapeDtypeStruct((n_nodes, 1), jnp.float32),
          jax.ShapeDtypeStruct((n_nodes, 1), jnp.float32),
          jax.ShapeDtypeStruct((n_nodes, d // 2), jnp.int32),
      ],
  )


def _final_body(a0_ref, a1_ref, d0_ref, d1_ref, lo_ref, nsq_ref,
                g_ref, xp_ref):
  agg = a0_ref[...] + a1_ref[...]
  den = d0_ref[...] + d1_ref[...]
  seg = agg / jnp.maximum(den, MIN_NORM)
  h = _proj(_expmap0(seg))
  xt = jnp.tanh(_logmap0(h))
  h = _proj(_expmap0(xt))
  h = _proj(h)
  lo_ref[...] = _logmap0(h)
  sq = jnp.sum(h * h, axis=-1, keepdims=True)
  nsq_ref[...] = sq
  n = jnp.maximum(jnp.sqrt(sq), MIN_NORM)
  g_ref[...] = _artanh(n) / n
  xp_ref[...] = _pack_rows(h)


@functools.lru_cache(maxsize=None)
def _make_final(n_nodes, d, bn):
  wide = pl.BlockSpec((bn, d), lambda i: (i, 0))
  thin = pl.BlockSpec((bn, 1), lambda i: (i, 0))
  return pl.pallas_call(
      _final_body,
      grid=(n_nodes // bn,),
      in_specs=[wide, wide, thin, thin],
      out_specs=[wide, thin, thin,
                 pl.BlockSpec((bn, d // 2), lambda i: (i, 0))],
      out_shape=[
          jax.ShapeDtypeStruct((n_nodes, d), jnp.float32),
          jax.ShapeDtypeStruct((n_nodes, 1), jnp.float32),
          jax.ShapeDtypeStruct((n_nodes, 1), jnp.float32),
          jax.ShapeDtypeStruct((n_nodes, d // 2), jnp.int32),
      ],
  )


def _out_body(cat_ref, o_ref):
  o_ref[...] = _proj(_expmap0(cat_ref[...]))


@functools.lru_cache(maxsize=None)
def _make_out(n_nodes, d, bn):
  return pl.pallas_call(
      _out_body,
      grid=(n_nodes // bn,),
      in_specs=[pl.BlockSpec((bn, d), lambda i: (i, 0))],
      out_specs=[pl.BlockSpec((bn, d), lambda i: (i, 0))],
      out_shape=[jax.ShapeDtypeStruct((n_nodes, d), jnp.float32)],
  )


@functools.lru_cache(maxsize=None)
def _make_sc_layer(n_nodes, d, nchunk, has_mask):
  mesh = plsc.VectorSubcoreMesh(core_axis_name="c", subcore_axis_name="s")
  n_grp = K // LANES
  n_unr = 8
  EPI = 4
  zr = 8
  per_tile = ((n_nodes // NS) // zr) * zr
  rem = n_nodes - NS * per_tile
  den_chunk = 640
  last = n_nodes - (NS - 1) * den_chunk

  def body(x_hbm, nsq_hbm, g_hbm, src_hbm, dst_hbm, msk_hbm,
           agg_out, den_out,
           sidx0, didx0, sidx1, didx1, srows0, drows0, srows1, drows1,
           a0, b0, gd0, m0, a1, b1, gd1, m1, wblk0, wblk1,
           ex_v, zrow_v, zden_v, agg_sh, den_sh, sem0, sem1, sem_w,
           sem_i0, sem_i1):
    cid = lax.axis_index("c")
    sid = lax.axis_index("s")
    wid = sid * NC + cid
    base = wid * (nchunk * K)

    zero16 = jnp.zeros((LANES,), jnp.float32)

    def zrow_body(r, carry):
      for col in range(d // LANES):
        zrow_v[r, pl.ds(col * LANES, LANES)] = zero16
      return carry

    lax.fori_loop(0, zr, zrow_body, 0)

    def zden_body(i, carry):
      zden_v[pl.ds(i * LANES, LANES)] = zero16
      return carry

    lax.fori_loop(0, den_chunk // LANES, zden_body, 0)

    def zagg_body(i, carry):
      pltpu.sync_copy(zrow_v, agg_sh.at[pl.ds(sid * per_tile + i * zr, zr)])
      return carry

    lax.fori_loop(0, per_tile // zr, zagg_body, 0)

    if rem:
      @pl.when(sid == NS - 1)
      def _():
        def zrem_body(i, carry):
          pltpu.sync_copy(zrow_v,
                          agg_sh.at[pl.ds(NS * per_tile + i * zr, zr)])
          return carry
        lax.fori_loop(0, rem // zr, zrem_body, 0)

    @pl.when(sid < NS - 1)
    def _():
      pltpu.sync_copy(zden_v, den_sh.at[pl.ds(sid * den_chunk, den_chunk)])

    @pl.when(sid == NS - 1)
    def _():
      pltpu.sync_copy(zden_v.at[pl.ds(0, last)],
                      den_sh.at[pl.ds((NS - 1) * den_chunk, last)])

    plsc.subcore_barrier()

    iot = lax.iota(jnp.int32, LANES)
    lanes = [jnp.full((LANES,), g * LANES, jnp.int32) + iot
             for g in range(n_grp)]

    bufs = [
        (sidx0, didx0, srows0, drows0, a0, b0, gd0, m0, sem0),
        (sidx1, didx1, srows1, drows1, a1, b1, gd1, m1, sem1),
    ]

    isems = [sem_i0, sem_i1]

    def fetch_idx_start(c, buf, isem):
      sidx, didx = buf[0], buf[1]
      off = base + c * K
      pltpu.async_copy(src_hbm.at[pl.ds(off, K)], sidx, isem)
      pltpu.async_copy(dst_hbm.at[pl.ds(off, K)], didx, isem)

    def fetch_idx_wait(c, buf, isem):
      sidx, didx = buf[0], buf[1]
      off = base + c * K
      pltpu.make_async_copy(src_hbm.at[pl.ds(off, K)], sidx, isem).wait()
      pltpu.make_async_copy(dst_hbm.at[pl.ds(off, K)], didx, isem).wait()

    def issue(c, buf):
      sidx, didx, srows, drows, a_b, b_b, gd_b, m_b, sem = buf
      pltpu.async_copy(x_hbm.at[sidx], srows, sem)
      pltpu.async_copy(x_hbm.at[didx], drows, sem)
      pltpu.async_copy(nsq_hbm.at[sidx], a_b, sem)
      pltpu.async_copy(nsq_hbm.at[didx], b_b, sem)
      pltpu.async_copy(g_hbm.at[didx], gd_b, sem)
      if has_mask:
        off = base + c * K
        pltpu.async_copy(msk_hbm.at[pl.ds(off, K)], m_b, sem)

    def drain(c, buf):
      sidx, didx, srows, drows, a_b, b_b, gd_b, m_b, sem = buf
      pltpu.make_async_copy(x_hbm.at[sidx], srows, sem).wait()
      pltpu.make_async_copy(x_hbm.at[didx], drows, sem).wait()
      pltpu.make_async_copy(nsq_hbm.at[sidx], a_b, sem).wait()
      pltpu.make_async_copy(nsq_hbm.at[didx], b_b, sem).wait()
      pltpu.make_async_copy(g_hbm.at[didx], gd_b, sem).wait()
      if has_mask:
        off = base + c * K
        pltpu.make_async_copy(msk_hbm.at[pl.ds(off, K)], m_b, sem).wait()

    rot = {sh: (iot + sh) & 15 for sh in (8, 4, 2, 1)}
    n_blk = (d // 2) // LANES
    zeros = jnp.zeros((LANES,), jnp.float32)

    def unpack_word(wv):
      return plsc.unpack(plsc.bitcast(wv, jnp.bfloat16),
                         format=plsc.PackFormat.INTERLEAVED)

    def compute(c, buf):
      sidx, didx, srows, drows, a_b, b_b, gd_b, m_b, sem = buf
      wbs = [wblk0, wblk1]
      s_ids_g = []
      for g in range(n_grp):
        e0 = g * LANES
        wb = wbs[g % 2]
        if g >= 2:
          pltpu.make_async_copy(wb, agg_sh.at[s_ids_g[g - 2]], sem_w).wait()

        def dot_pair(pi, dots, e0=e0):
          res = dots
          loads = []
          for sub in range(EPI):
            e = e0 + pi * EPI + sub
            sb = [srows[e, pl.ds(blk * LANES, LANES)] for blk in range(n_blk)]
            db = [drows[e, pl.ds(blk * LANES, LANES)] for blk in range(n_blk)]
            loads.append((sb, db))
          for sub in range(EPI):
            sb, db = loads[sub]
            ms = []
            for sv, dv in zip(sb, db):
              slo, shi = unpack_word(sv)
              dlo, dhi = unpack_word(dv)
              ms.append(slo * dlo)
              ms.append(shi * dhi)
            while len(ms) > 1:
              ms = [ms[i] + ms[i + 1] for i in range(0, len(ms), 2)]
            r = ms[0]
            for sh in (8, 4, 2, 1):
              r = r + _vperm(r, rot[sh])
            res = jnp.where(iot == pi * EPI + sub, r, res)
          return res

        dots = lax.fori_loop(0, LANES // EPI, dot_pair, zeros)

        a = a_b[pl.ds(e0, LANES)]
        b = b_b[pl.ds(e0, LANES)]
        A = 1.0 - 2.0 * dots + b
        B = 1.0 - a
        num2 = A * A * a - 2.0 * A * B * dots + B * B * b
        den = 1.0 - 2.0 * dots + a * b
        norm = _sw_sqrt(jnp.maximum(num2, 0.0)) / jnp.maximum(den, MIN_NORM)
        z = jnp.minimum(norm, 1.0 - 1e-7)
        dist = _sw_ln((1.0 + z) / (1.0 - z))
        ex = jnp.exp(dist * dist)
        if has_mask:
          ex = ex * m_b[pl.ds(e0, LANES)]
        ex_v[pl.ds(e0, LANES)] = ex
        w_vec = ex * gd_b[pl.ds(e0, LANES)]
        s_ids_g.append(sidx[pl.ds(e0, LANES)])

        def scale_pair(pi, carry, e0=e0, wb=wb, w_vec=w_vec):
          ws = [_vperm(w_vec, jnp.full((LANES,), pi * EPI + s, jnp.int32))
                for s in range(EPI)]
          loads = []
          for sub in range(EPI):
            e = e0 + pi * EPI + sub
            loads.append([drows[e, pl.ds(blk * LANES, LANES)]
                          for blk in range(n_blk)])
          for sub in range(EPI):
            ei = pi * EPI + sub
            for blk in range(n_blk):
              lo, hi = unpack_word(loads[sub][blk])
              wb[ei, pl.ds(blk * LANES, LANES)] = lo * ws[sub]
              wb[ei, pl.ds(d // 2 + blk * LANES, LANES)] = hi * ws[sub]
          return carry

        lax.fori_loop(0, LANES // EPI, scale_pair, 0)
        pltpu.async_copy(wb, agg_sh.at[s_ids_g[g]], sem_w, add=True)
      for g in range(max(0, n_grp - 2), n_grp):
        pltpu.make_async_copy(wbs[g % 2], agg_sh.at[s_ids_g[g]], sem_w).wait()

      pltpu.sync_copy(ex_v, den_sh.at[sidx], add=True)

    fetch_idx_start(0, bufs[0], isems[0])
    fetch_idx_wait(0, bufs[0], isems[0])
    issue(0, bufs[0])
    if nchunk > 1:
      fetch_idx_start(1, bufs[1], isems[1])

    def pair_body(i, carry):
      c0 = 2 * i
      c1 = c0 + 1
      fetch_idx_wait(c1, bufs[1], isems[1])
      issue(c1, bufs[1])
      drain(c0, bufs[0])
      compute(c0, bufs[0])

      @pl.when(c0 + 2 < nchunk)
      def _():
        fetch_idx_start(c0 + 2, bufs[0], isems[0])

      drain(c1, bufs[1])

      @pl.when(c0 + 2 < nchunk)
      def _():
        fetch_idx_wait(c0 + 2, bufs[0], isems[0])
        issue(c0 + 2, bufs[0])

      compute(c1, bufs[1])

      @pl.when(c0 + 3 < nchunk)
      def _():
        fetch_idx_start(c0 + 3, bufs[1], isems[1])

      return carry

    lax.fori_loop(0, nchunk // 2, pair_body, 0)

    if nchunk % 2:
      c_last = nchunk - 1
      drain(c_last, bufs[0])
      compute(c_last, bufs[0])

    plsc.subcore_barrier()

    def co_body(i, carry):
      r0 = sid * per_tile + i * zr
      pltpu.sync_copy(agg_sh.at[pl.ds(r0, zr)],
                      agg_out.at[pl.ds(cid * n_nodes + r0, zr)])
      return carry

    lax.fori_loop(0, per_tile // zr, co_body, 0)

    if rem:
      @pl.when(sid == NS - 1)
      def _():
        def corem_body(i, carry):
          r0 = NS * per_tile + i * zr
          pltpu.sync_copy(agg_sh.at[pl.ds(r0, zr)],
                          agg_out.at[pl.ds(cid * n_nodes + r0, zr)])
          return carry
        lax.fori_loop(0, rem // zr, corem_body, 0)

    @pl.when(sid < NS - 1)
    def _():
      pltpu.sync_copy(den_sh.at[pl.ds(sid * den_chunk, den_chunk)], zden_v)
      pltpu.sync_copy(
          zden_v,
          den_out.at[pl.ds(cid * n_nodes + sid * den_chunk, den_chunk)])

    @pl.when(sid == NS - 1)
    def _():
      pltpu.sync_copy(den_sh.at[pl.ds((NS - 1) * den_chunk, last)],
                      zden_v.at[pl.ds(0, last)])
      pltpu.sync_copy(
          zden_v.at[pl.ds(0, last)],
          den_out.at[pl.ds(cid * n_nodes + (NS - 1) * den_chunk, last)])

  return pl.kernel(
      body,
      out_type=[
          jax.ShapeDtypeStruct((NC * n_nodes, d), jnp.float32),
          jax.ShapeDtypeStruct((NC * n_nodes,), jnp.float32),
      ],
      mesh=mesh,
      compiler_params=pltpu.CompilerParams(needs_layout_passes=False,
                                           use_tc_tiling_on_sc=False),
      scratch_types=[
          pltpu.VMEM((K,), jnp.int32),
          pltpu.VMEM((K,), jnp.int32),
          pltpu.VMEM((K,), jnp.int32),
          pltpu.VMEM((K,), jnp.int32),
          pltpu.VMEM((K, d // 2), jnp.int32),
          pltpu.VMEM((K, d // 2), jnp.int32),
          pltpu.VMEM((K, d // 2), jnp.int32),
          pltpu.VMEM((K, d // 2), jnp.int32),
          pltpu.VMEM((K,), jnp.float32),
          pltpu.VMEM((K,), jnp.float32),
          pltpu.VMEM((K,), jnp.float32),
          pltpu.VMEM((K,), jnp.float32),
          pltpu.VMEM((K,), jnp.float32),
          pltpu.VMEM((K,), jnp.float32),
          pltpu.VMEM((K,), jnp.float32),
          pltpu.VMEM((K,), jnp.float32),
          pltpu.VMEM((LANES, d), jnp.float32),
          pltpu.VMEM((LANES, d), jnp.float32),
          pltpu.VMEM((K,), jnp.float32),
          pltpu.VMEM((zr, d), jnp.float32),
          pltpu.VMEM((640,), jnp.float32),
          pltpu.VMEM_SHARED((n_nodes, d), jnp.float32),
          pltpu.VMEM_SHARED((n_nodes,), jnp.float32),
          pltpu.SemaphoreType.DMA,
          pltpu.SemaphoreType.DMA,
          pltpu.SemaphoreType.DMA,
          pltpu.SemaphoreType.DMA,
          pltpu.SemaphoreType.DMA,
      ],
  )


@jax.jit
def kernel(input, edge_index):
  x0 = input.astype(jnp.float32)
  n_nodes, d = x0.shape
  e = edge_index.shape[1]

  blk = NW * K
  ep = ((e + blk - 1) // blk) * blk
  src = edge_index[0]
  dst = edge_index[1]
  if ep != e:
    src = jnp.concatenate([src, jnp.zeros((ep - e,), jnp.int32)])
    dst = jnp.concatenate([dst, jnp.zeros((ep - e,), jnp.int32)])
    msk = (jnp.arange(ep) < e).astype(jnp.float32)
  else:
    msk = jnp.ones((ep,), jnp.float32)
  ew = ep // NW
  nchunk = ew // K
  sc_layer = _make_sc_layer(n_nodes, d, nchunk, ep != e)
  bn = 1000 if n_nodes % 1000 == 0 else 8
  prep = _make_prep(n_nodes, d, bn)
  final = _make_final(n_nodes, d, bn)

  nsq, gfac, xp = prep(x0)
  nsq_flat = nsq.reshape((n_nodes,))
  g_flat = gfac.reshape((n_nodes,))

  outs = [x0]
  for _ in range(2):
    agg_p, den_p = sc_layer(xp, nsq_flat, g_flat, src, dst, msk)
    lo, nsq, gfac, xp = final(agg_p[:n_nodes], agg_p[n_nodes:],
                              den_p[:n_nodes].reshape((n_nodes, 1)),
                              den_p[n_nodes:].reshape((n_nodes, 1)))
    nsq_flat = nsq.reshape((n_nodes,))
    g_flat = gfac.reshape((n_nodes,))
    outs.append(lo)

  cat = jnp.concatenate(outs, axis=-1)
  out_tc = _make_out(n_nodes, cat.shape[1], bn)
  (out,) = out_tc(cat)
  return out


# speedup vs baseline: 1.1368x; 1.1368x over previous
import functools

import jax
import jax.numpy as jnp
from jax import lax
from jax.experimental import pallas as pl
from jax.experimental.pallas import tpu as pltpu
from jax.experimental.pallas import tpu_sc as plsc

MIN_NORM = 1e-15
NC, NS = 2, 16
NW = NC * NS
LANES = 16
K = 80
LN2 = 0.6931471805599453


def _artanh(z):
  z = jnp.clip(z, -1.0 + 1e-7, 1.0 - 1e-7)
  return 0.5 * jnp.log((1.0 + z) / (1.0 - z))


def _rownorm(v):
  return jnp.sqrt(jnp.sum(v * v, axis=-1, keepdims=True))


def _logmap0(p):
  n = jnp.maximum(_rownorm(p), MIN_NORM)
  return p / n * _artanh(n)


def _expmap0(u):
  n = jnp.maximum(_rownorm(u), MIN_NORM)
  return jnp.tanh(n) * u / n


def _proj(x):
  n = jnp.maximum(_rownorm(x), MIN_NORM)
  maxn = 1.0 - 4e-3
  return jnp.where(n > maxn, x / n * maxn, x)


def _vperm(v, idx):
  dnums = lax.GatherDimensionNumbers(
      offset_dims=(), collapsed_slice_dims=(0,), start_index_map=(0,))
  return lax.gather(v, idx[:, None], dnums, (1,),
                    mode=lax.GatherScatterMode.PROMISE_IN_BOUNDS)


def _sw_rsqrt(x):
  i = lax.bitcast_convert_type(x, jnp.int32)
  i = 0x5F3759DF - lax.shift_right_logical(i, 1)
  y = lax.bitcast_convert_type(i, jnp.float32)
  for _ in range(3):
    y = y * (1.5 - 0.5 * x * y * y)
  return y


def _sw_sqrt(x):
  return jnp.where(x < 1e-30, 0.0, x * _sw_rsqrt(x))


def _sw_ln(y):
  bits = lax.bitcast_convert_type(y, jnp.int32)
  e = lax.shift_right_logical(bits, 23) - 127
  m = lax.bitcast_convert_type((bits & 0x007FFFFF) | 0x3F800000, jnp.float32)
  big = m > 1.4142135
  m = jnp.where(big, m * 0.5, m)
  e = (e + jnp.where(big, 1, 0)).astype(jnp.float32)
  s = (m - 1.0) / (m + 1.0)
  s2 = s * s
  p = 1.0 / 9.0
  p = p * s2 + 1.0 / 7.0
  p = p * s2 + 1.0 / 5.0
  p = p * s2 + 1.0 / 3.0
  p = p * s2 + 1.0
  return e * LN2 + 2.0 * s * p


def _pack_rows(x):
  d = x.shape[-1]
  xb = x.astype(jnp.bfloat16)
  lo = lax.bitcast_convert_type(xb[:, :d // 2], jnp.uint16).astype(jnp.uint32)
  hi = lax.bitcast_convert_type(xb[:, d // 2:], jnp.uint16).astype(jnp.uint32)
  return lax.bitcast_convert_type((hi << 16) | lo, jnp.int32)


def _prep_body(x_ref, nsq_ref, g_ref, xp_ref):
  x = x_ref[...]
  sq = jnp.sum(x * x, axis=-1, keepdims=True)
  nsq_ref[...] = sq
  n = jnp.maximum(jnp.sqrt(sq), MIN_NORM)
  g_ref[...] = _artanh(n) / n
  xp_ref[...] = _pack_rows(x)


@functools.lru_cache(maxsize=None)
def _make_prep(n_nodes, d, bn):
  return pl.pallas_call(
      _prep_body,
      grid=(n_nodes // bn,),
      in_specs=[pl.BlockSpec((bn, d), lambda i: (i, 0))],
      out_specs=[
          pl.BlockSpec((bn, 1), lambda i: (i, 0)),
          pl.BlockSpec((bn, 1), lambda i: (i, 0)),
          pl.BlockSpec((bn, d // 2), lambda i: (i, 0)),
      ],
      out_shape=[
          jax.ShapeDtypeStruct((n_nodes, 1), jnp.float32),
          jax.ShapeDtypeStruct((n_nodes, 1), jnp.float32),
          jax.ShapeDtypeStruct((n_nodes, d // 2), jnp.int32),
      ],
  )


def _final_body(a0_ref, a1_ref, d0_ref, d1_ref, lo_ref, nsq_ref,
                g_ref, xp_ref):
  agg = a0_ref[...] + a1_ref[...]
  den = d0_ref[...] + d1_ref[...]
  seg = agg / jnp.maximum(den, MIN_NORM)
  h = _proj(_expmap0(seg))
  xt = jnp.tanh(_logmap0(h))
  h = _proj(_expmap0(xt))
  h = _proj(h)
  lo_ref[...] = _logmap0(h)
  sq = jnp.sum(h * h, axis=-1, keepdims=True)
  nsq_ref[...] = sq
  n = jnp.maximum(jnp.sqrt(sq), MIN_NORM)
  g_ref[...] = _artanh(n) / n
  xp_ref[...] = _pack_rows(h)


@functools.lru_cache(maxsize=None)
def _make_final(n_nodes, d, bn):
  wide = pl.BlockSpec((bn, d), lambda i: (i, 0))
  thin = pl.BlockSpec((bn, 1), lambda i: (i, 0))
  return pl.pallas_call(
      _final_body,
      grid=(n_nodes // bn,),
      in_specs=[wide, wide, thin, thin],
      out_specs=[wide, thin, thin,
                 pl.BlockSpec((bn, d // 2), lambda i: (i, 0))],
      out_shape=[
          jax.ShapeDtypeStruct((n_nodes, d), jnp.float32),
          jax.ShapeDtypeStruct((n_nodes, 1), jnp.float32),
          jax.ShapeDtypeStruct((n_nodes, 1), jnp.float32),
          jax.ShapeDtypeStruct((n_nodes, d // 2), jnp.int32),
      ],
  )


def _out_body(cat_ref, o_ref):
  o_ref[...] = _proj(_expmap0(cat_ref[...]))


@functools.lru_cache(maxsize=None)
def _make_out(n_nodes, d, bn):
  return pl.pallas_call(
      _out_body,
      grid=(n_nodes // bn,),
      in_specs=[pl.BlockSpec((bn, d), lambda i: (i, 0))],
      out_specs=[pl.BlockSpec((bn, d), lambda i: (i, 0))],
      out_shape=[jax.ShapeDtypeStruct((n_nodes, d), jnp.float32)],
  )


@functools.lru_cache(maxsize=None)
def _make_sc_layer(n_nodes, d, nchunk, has_mask):
  mesh = plsc.VectorSubcoreMesh(core_axis_name="c", subcore_axis_name="s")
  n_grp = K // LANES
  n_unr = 8
  zr = 8
  per_tile = ((n_nodes // NS) // zr) * zr
  rem = n_nodes - NS * per_tile
  den_chunk = 640
  last = n_nodes - (NS - 1) * den_chunk

  def body(x_hbm, nsq_hbm, g_hbm, src_hbm, dst_hbm, msk_hbm,
           agg_out, den_out,
           sidx0, didx0, sidx1, didx1, srows0, drows0, srows1, drows1,
           a0, b0, gd0, m0, a1, b1, gd1, m1, wblk0, wblk1,
           ex_v, zrow_v, zden_v, agg_sh, den_sh, sem0, sem1, sem_w,
           sem_i0, sem_i1):
    cid = lax.axis_index("c")
    sid = lax.axis_index("s")
    wid = sid * NC + cid
    base = wid * (nchunk * K)

    zero16 = jnp.zeros((LANES,), jnp.float32)

    def zrow_body(r, carry):
      for col in range(d // LANES):
        zrow_v[r, pl.ds(col * LANES, LANES)] = zero16
      return carry

    lax.fori_loop(0, zr, zrow_body, 0)

    def zden_body(i, carry):
      zden_v[pl.ds(i * LANES, LANES)] = zero16
      return carry

    lax.fori_loop(0, den_chunk // LANES, zden_body, 0)

    def zagg_body(i, carry):
      pltpu.sync_copy(zrow_v, agg_sh.at[pl.ds(sid * per_tile + i * zr, zr)])
      return carry

    lax.fori_loop(0, per_tile // zr, zagg_body, 0)

    if rem:
      @pl.when(sid == NS - 1)
      def _():
        def zrem_body(i, carry):
          pltpu.sync_copy(zrow_v,
                          agg_sh.at[pl.ds(NS * per_tile + i * zr, zr)])
          return carry
        lax.fori_loop(0, rem // zr, zrem_body, 0)

    @pl.when(sid < NS - 1)
    def _():
      pltpu.sync_copy(zden_v, den_sh.at[pl.ds(sid * den_chunk, den_chunk)])

    @pl.when(sid == NS - 1)
    def _():
      pltpu.sync_copy(zden_v.at[pl.ds(0, last)],
                      den_sh.at[pl.ds((NS - 1) * den_chunk, last)])

    plsc.subcore_barrier()

    iot = lax.iota(jnp.int32, LANES)
    lanes = [jnp.full((LANES,), g * LANES, jnp.int32) + iot
             for g in range(n_grp)]

    bufs = [
        (sidx0, didx0, srows0, drows0, a0, b0, gd0, m0, sem0),
        (sidx1, didx1, srows1, drows1, a1, b1, gd1, m1, sem1),
    ]

    isems = [sem_i0, sem_i1]

    def fetch_idx_start(c, buf, isem):
      sidx, didx = buf[0], buf[1]
      off = base + c * K
      pltpu.async_copy(src_hbm.at[pl.ds(off, K)], sidx, isem)
      pltpu.async_copy(dst_hbm.at[pl.ds(off, K)], didx, isem)

    def fetch_idx_wait(c, buf, isem):
      sidx, didx = buf[0], buf[1]
      off = base + c * K
      pltpu.make_async_copy(src_hbm.at[pl.ds(off, K)], sidx, isem).wait()
      pltpu.make_async_copy(dst_hbm.at[pl.ds(off, K)], didx, isem).wait()

    def issue(c, buf):
      sidx, didx, srows, drows, a_b, b_b, gd_b, m_b, sem = buf
      pltpu.async_copy(x_hbm.at[sidx], srows, sem)
      pltpu.async_copy(x_hbm.at[didx], drows, sem)
      pltpu.async_copy(nsq_hbm.at[sidx], a_b, sem)
      pltpu.async_copy(nsq_hbm.at[didx], b_b, sem)
      pltpu.async_copy(g_hbm.at[didx], gd_b, sem)
      if has_mask:
        off = base + c * K
        pltpu.async_copy(msk_hbm.at[pl.ds(off, K)], m_b, sem)

    def drain(c, buf):
      sidx, didx, srows, drows, a_b, b_b, gd_b, m_b, sem = buf
      pltpu.make_async_copy(x_hbm.at[sidx], srows, sem).wait()
      pltpu.make_async_copy(x_hbm.at[didx], drows, sem).wait()
      pltpu.make_async_copy(nsq_hbm.at[sidx], a_b, sem).wait()
      pltpu.make_async_copy(nsq_hbm.at[didx], b_b, sem).wait()
      pltpu.make_async_copy(g_hbm.at[didx], gd_b, sem).wait()
      if has_mask:
        off = base + c * K
        pltpu.make_async_copy(msk_hbm.at[pl.ds(off, K)], m_b, sem).wait()

    rot = {sh: (iot + sh) & 15 for sh in (8, 4, 2, 1)}
    n_blk = (d // 2) // LANES
    zeros = jnp.zeros((LANES,), jnp.float32)

    def unpack_word(wv):
      return plsc.unpack(plsc.bitcast(wv, jnp.bfloat16),
                         format=plsc.PackFormat.INTERLEAVED)

    def compute(c, buf):
      sidx, didx, srows, drows, a_b, b_b, gd_b, m_b, sem = buf
      wbs = [wblk0, wblk1]
      s_ids_g = []
      for g in range(n_grp):
        e0 = g * LANES
        wb = wbs[g % 2]
        if g >= 2:
          pltpu.make_async_copy(wb, agg_sh.at[s_ids_g[g - 2]], sem_w).wait()

        def dot_pair(pi, dots, e0=e0):
          res = dots
          loads = []
          for sub in range(2):
            e = e0 + pi * 2 + sub
            sb = [srows[e, pl.ds(blk * LANES, LANES)] for blk in range(n_blk)]
            db = [drows[e, pl.ds(blk * LANES, LANES)] for blk in range(n_blk)]
            loads.append((sb, db))
          for sub in range(2):
            sb, db = loads[sub]
            ms = []
            for sv, dv in zip(sb, db):
              slo, shi = unpack_word(sv)
              dlo, dhi = unpack_word(dv)
              ms.append(slo * dlo)
              ms.append(shi * dhi)
            while len(ms) > 1:
              ms = [ms[i] + ms[i + 1] for i in range(0, len(ms), 2)]
            r = ms[0]
            for sh in (8, 4, 2, 1):
              r = r + _vperm(r, rot[sh])
            res = jnp.where(iot == pi * 2 + sub, r, res)
          return res

        dots = lax.fori_loop(0, LANES // 2, dot_pair, zeros)

        a = a_b[pl.ds(e0, LANES)]
        b = b_b[pl.ds(e0, LANES)]
        A = 1.0 - 2.0 * dots + b
        B = 1.0 - a
        num2 = A * A * a - 2.0 * A * B * dots + B * B * b
        den = 1.0 - 2.0 * dots + a * b
        norm = _sw_sqrt(jnp.maximum(num2, 0.0)) / jnp.maximum(den, MIN_NORM)
        z = jnp.minimum(norm, 1.0 - 1e-7)
        dist = _sw_ln((1.0 + z) / (1.0 - z))
        ex = jnp.exp(dist * dist)
        if has_mask:
          ex = ex * m_b[pl.ds(e0, LANES)]
        ex_v[pl.ds(e0, LANES)] = ex
        w_vec = ex * gd_b[pl.ds(e0, LANES)]
        s_ids_g.append(sidx[pl.ds(e0, LANES)])

        def scale_pair(pi, carry, e0=e0, wb=wb, w_vec=w_vec):
          ws = [_vperm(w_vec, jnp.full((LANES,), pi * 2 + s, jnp.int32))
                for s in range(2)]
          loads = []
          for sub in range(2):
            e = e0 + pi * 2 + sub
            loads.append([drows[e, pl.ds(blk * LANES, LANES)]
                          for blk in range(n_blk)])
          for sub in range(2):
            ei = pi * 2 + sub
            for blk in range(n_blk):
              lo, hi = unpack_word(loads[sub][blk])
              wb[ei, pl.ds(blk * LANES, LANES)] = lo * ws[sub]
              wb[ei, pl.ds(d // 2 + blk * LANES, LANES)] = hi * ws[sub]
          return carry

        lax.fori_loop(0, LANES // 2, scale_pair, 0)
        pltpu.async_copy(wb, agg_sh.at[s_ids_g[g]], sem_w, add=True)
      for g in range(max(0, n_grp - 2), n_grp):
        pltpu.make_async_copy(wbs[g % 2], agg_sh.at[s_ids_g[g]], sem_w).wait()

      pltpu.sync_copy(ex_v, den_sh.at[sidx], add=True)

    fetch_idx_start(0, bufs[0], isems[0])
    fetch_idx_wait(0, bufs[0], isems[0])
    issue(0, bufs[0])
    if nchunk > 1:
      fetch_idx_start(1, bufs[1], isems[1])

    def pair_body(i, carry):
      c0 = 2 * i
      c1 = c0 + 1
      fetch_idx_wait(c1, bufs[1], isems[1])
      issue(c1, bufs[1])
      drain(c0, bufs[0])
      compute(c0, bufs[0])

      @pl.when(c0 + 2 < nchunk)
      def _():
        fetch_idx_start(c0 + 2, bufs[0], isems[0])

      drain(c1, bufs[1])

      @pl.when(c0 + 2 < nchunk)
      def _():
        fetch_idx_wait(c0 + 2, bufs[0], isems[0])
        issue(c0 + 2, bufs[0])

      compute(c1, bufs[1])

      @pl.when(c0 + 3 < nchunk)
      def _():
        fetch_idx_start(c0 + 3, bufs[1], isems[1])

      return carry

    lax.fori_loop(0, nchunk // 2, pair_body, 0)

    if nchunk % 2:
      c_last = nchunk - 1
      drain(c_last, bufs[0])
      compute(c_last, bufs[0])

    plsc.subcore_barrier()

    def co_body(i, carry):
      r0 = sid * per_tile + i * zr
      pltpu.sync_copy(agg_sh.at[pl.ds(r0, zr)],
                      agg_out.at[pl.ds(cid * n_nodes + r0, zr)])
      return carry

    lax.fori_loop(0, per_tile // zr, co_body, 0)

    if rem:
      @pl.when(sid == NS - 1)
      def _():
        def corem_body(i, carry):
          r0 = NS * per_tile + i * zr
          pltpu.sync_copy(agg_sh.at[pl.ds(r0, zr)],
                          agg_out.at[pl.ds(cid * n_nodes + r0, zr)])
          return carry
        lax.fori_loop(0, rem // zr, corem_body, 0)

    @pl.when(sid < NS - 1)
    def _():
      pltpu.sync_copy(den_sh.at[pl.ds(sid * den_chunk, den_chunk)], zden_v)
      pltpu.sync_copy(
          zden_v,
          den_out.at[pl.ds(cid * n_nodes + sid * den_chunk, den_chunk)])

    @pl.when(sid == NS - 1)
    def _():
      pltpu.sync_copy(den_sh.at[pl.ds((NS - 1) * den_chunk, last)],
                      zden_v.at[pl.ds(0, last)])
      pltpu.sync_copy(
          zden_v.at[pl.ds(0, last)],
          den_out.at[pl.ds(cid * n_nodes + (NS - 1) * den_chunk, last)])

  return pl.kernel(
      body,
      out_type=[
          jax.ShapeDtypeStruct((NC * n_nodes, d), jnp.float32),
          jax.ShapeDtypeStruct((NC * n_nodes,), jnp.float32),
      ],
      mesh=mesh,
      compiler_params=pltpu.CompilerParams(needs_layout_passes=False,
                                           use_tc_tiling_on_sc=False),
      scratch_types=[
          pltpu.VMEM((K,), jnp.int32),
          pltpu.VMEM((K,), jnp.int32),
          pltpu.VMEM((K,), jnp.int32),
          pltpu.VMEM((K,), jnp.int32),
          pltpu.VMEM((K, d // 2), jnp.int32),
          pltpu.VMEM((K, d // 2), jnp.int32),
          pltpu.VMEM((K, d // 2), jnp.int32),
          pltpu.VMEM((K, d // 2), jnp.int32),
          pltpu.VMEM((K,), jnp.float32),
          pltpu.VMEM((K,), jnp.float32),
          pltpu.VMEM((K,), jnp.float32),
          pltpu.VMEM((K,), jnp.float32),
          pltpu.VMEM((K,), jnp.float32),
          pltpu.VMEM((K,), jnp.float32),
          pltpu.VMEM((K,), jnp.float32),
          pltpu.VMEM((K,), jnp.float32),
          pltpu.VMEM((LANES, d), jnp.float32),
          pltpu.VMEM((LANES, d), jnp.float32),
          pltpu.VMEM((K,), jnp.float32),
          pltpu.VMEM((zr, d), jnp.float32),
          pltpu.VMEM((640,), jnp.float32),
          pltpu.VMEM_SHARED((n_nodes, d), jnp.float32),
          pltpu.VMEM_SHARED((n_nodes,), jnp.float32),
          pltpu.SemaphoreType.DMA,
          pltpu.SemaphoreType.DMA,
          pltpu.SemaphoreType.DMA,
          pltpu.SemaphoreType.DMA,
          pltpu.SemaphoreType.DMA,
      ],
  )


@jax.jit
def kernel(input, edge_index):
  x0 = input.astype(jnp.float32)
  n_nodes, d = x0.shape
  e = edge_index.shape[1]

  blk = NW * K
  ep = ((e + blk - 1) // blk) * blk
  src = edge_index[0]
  dst = edge_index[1]
  if ep != e:
    src = jnp.concatenate([src, jnp.zeros((ep - e,), jnp.int32)])
    dst = jnp.concatenate([dst, jnp.zeros((ep - e,), jnp.int32)])
    msk = (jnp.arange(ep) < e).astype(jnp.float32)
  else:
    msk = jnp.ones((ep,), jnp.float32)
  ew = ep // NW
  nchunk = ew // K
  sc_layer = _make_sc_layer(n_nodes, d, nchunk, ep != e)
  bn = 1000 if n_nodes % 1000 == 0 else 8
  prep = _make_prep(n_nodes, d, bn)
  final = _make_final(n_nodes, d, bn)

  nsq, gfac, xp = prep(x0)
  nsq_flat = nsq.reshape((n_nodes,))
  g_flat = gfac.reshape((n_nodes,))

  outs = [x0]
  for _ in range(2):
    agg_p, den_p = sc_layer(xp, nsq_flat, g_flat, src, dst, msk)
    lo, nsq, gfac, xp = final(agg_p[:n_nodes], agg_p[n_nodes:],
                              den_p[:n_nodes].reshape((n_nodes, 1)),
                              den_p[n_nodes:].reshape((n_nodes, 1)))
    nsq_flat = nsq.reshape((n_nodes,))
    g_flat = gfac.reshape((n_nodes,))
    outs.append(lo)

  cat = jnp.concatenate(outs, axis=-1)
  out_tc = _make_out(n_nodes, cat.shape[1], bn)
  (out,) = out_tc(cat)
  return out
